# Initial kernel scaffold; baseline (speedup 1.0000x reference)
import jax
import jax.numpy as jnp
from jax.experimental import pallas as pl


def kernel(features, edge_index, edge_weight, W, b):
    raise NotImplementedError("write your pallas kernel here")


# speedup vs baseline: 4.3968x; 4.3968x over previous
import functools

import jax
import jax.numpy as jnp
from jax import lax
from jax.experimental import pallas as pl
from jax.experimental.pallas import tpu as pltpu
from jax.experimental.pallas import tpu_sc as plsc

N_NODES = 10000
N_EDGES = 320000
D = 128

NC = 2
NS = 16
NW = NC * NS

EPT = N_EDGES // NW
C = 80
NCHUNK = EPT // C
RPT = 624
TAIL = N_NODES - NS * RPT
ZR = 48


def _vbroadcast(vec, k):
    idx = jnp.full((16,), k, jnp.int32)
    return lax.gather(
        vec, idx[:, None],
        lax.GatherDimensionNumbers(
            offset_dims=(), collapsed_slice_dims=(0,), start_index_map=(0,)),
        slice_sizes=(1,),
        mode=lax.GatherScatterMode.PROMISE_IN_BOUNDS)


def _mm_body(x_ref, w_ref, o_ref):
    o_ref[...] = jnp.dot(x_ref[...], w_ref[...],
                         preferred_element_type=jnp.float32)


def _matmul(features, W):
    return pl.pallas_call(
        _mm_body,
        grid=(10,),
        in_specs=[
            pl.BlockSpec((N_NODES // 10, D), lambda i: (i, 0)),
            pl.BlockSpec((D, D), lambda i: (0, 0)),
        ],
        out_specs=pl.BlockSpec((N_NODES // 10, D), lambda i: (i, 0)),
        out_shape=jax.ShapeDtypeStruct((N_NODES, D), jnp.float32),
    )(features, W)


def _sc_agg_body(h_hbm, src_hbm, dst_hbm, ew_hbm, out_hbm,
                 acc, src_v, dst_v, ew_v, rows_v, zero_v, sem):
    cid = lax.axis_index("c")
    sid = lax.axis_index("s")
    tid = sid * NC + cid

    def zfill(r, carry):
        for j in range(D // 16):
            zero_v[r, pl.ds(j * 16, 16)] = jnp.zeros((16,), jnp.float32)
        return carry
    lax.fori_loop(0, ZR, zfill, 0)

    def zcopy(i, carry):
        pltpu.sync_copy(zero_v, acc.at[pl.ds(sid * RPT + i * ZR, ZR)])
        return carry
    lax.fori_loop(0, RPT // ZR, zcopy, 0)

    @pl.when(sid == NS - 1)
    def _():
        pltpu.sync_copy(zero_v.at[pl.ds(0, TAIL)],
                        acc.at[pl.ds(NS * RPT, TAIL)])
    plsc.subcore_barrier()

    ebase = tid * EPT

    def chunk(ci, carry):
        off = ebase + ci * C
        pltpu.sync_copy(src_hbm.at[pl.ds(off, C)], src_v)
        pltpu.sync_copy(dst_hbm.at[pl.ds(off, C)], dst_v)
        pltpu.sync_copy(ew_hbm.at[pl.ds(off, C)], ew_v)
        pltpu.async_copy(h_hbm.at[src_v], rows_v, sem).wait()

        def scale_group(g, c2):
            wvec = ew_v[pl.ds(g * 16, 16)]

            def scale_edge(k, c3):
                w = _vbroadcast(wvec, k)
                e = g * 16 + k
                for j in range(D // 16):
                    sl = pl.ds(j * 16, 16)
                    rows_v[e, sl] = rows_v[e, sl] * w
                return c3
            return lax.fori_loop(0, 16, scale_edge, c2)
        lax.fori_loop(0, C // 16, scale_group, 0)

        pltpu.sync_copy(rows_v, acc.at[dst_v], add=True)
        return carry
    lax.fori_loop(0, NCHUNK, chunk, 0)
    plsc.subcore_barrier()

    rbase = sid * RPT
    pltpu.sync_copy(acc.at[pl.ds(rbase, RPT)],
                    out_hbm.at[cid, pl.ds(rbase, RPT)])

    @pl.when(sid == NS - 1)
    def _():
        pltpu.sync_copy(acc.at[pl.ds(NS * RPT, TAIL)],
                        out_hbm.at[cid, pl.ds(NS * RPT, TAIL)])


_sc_agg = functools.partial(
    pl.kernel,
    out_type=jax.ShapeDtypeStruct((NC, N_NODES, D), jnp.float32),
    mesh=plsc.VectorSubcoreMesh(core_axis_name="c", subcore_axis_name="s"),
    scratch_types=[
        pltpu.VMEM_SHARED((N_NODES, D), jnp.float32),
        pltpu.VMEM((C,), jnp.int32),
        pltpu.VMEM((C,), jnp.int32),
        pltpu.VMEM((C,), jnp.float32),
        pltpu.VMEM((C, D), jnp.float32),
        pltpu.VMEM((ZR, D), jnp.float32),
        pltpu.SemaphoreType.DMA,
    ],
)(_sc_agg_body)


def _comb_body(p0_ref, p1_ref, b_ref, o_ref):
    o_ref[...] = p0_ref[...] + p1_ref[...] + b_ref[...]


def _combine(partials, b):
    return pl.pallas_call(
        _comb_body,
        grid=(10,),
        in_specs=[
            pl.BlockSpec((N_NODES // 10, D), lambda i: (i, 0)),
            pl.BlockSpec((N_NODES // 10, D), lambda i: (i, 0)),
            pl.BlockSpec((1, D), lambda i: (0, 0)),
        ],
        out_specs=pl.BlockSpec((N_NODES // 10, D), lambda i: (i, 0)),
        out_shape=jax.ShapeDtypeStruct((N_NODES, D), jnp.float32),
    )(partials[0], partials[1], b.reshape(1, D))


@jax.jit
def kernel(features, edge_index, edge_weight, W, b):
    h = _matmul(features, W)
    partials = _sc_agg(h, edge_index[0], edge_index[1], edge_weight)
    return _combine(partials, b)


# speedup vs baseline: 10.3653x; 2.3575x over previous
import functools

import jax
import jax.numpy as jnp
from jax import lax
from jax.experimental import pallas as pl
from jax.experimental.pallas import tpu as pltpu
from jax.experimental.pallas import tpu_sc as plsc

N_NODES = 10000
N_EDGES = 320000
D = 128

NC = 2
NS = 16
NW = NC * NS

EPT = N_EDGES // NW
C = 80
NCHUNK = EPT // C
RPT = 624
TAIL = N_NODES - NS * RPT


def _vbroadcast(vec, k):
    idx = jnp.full((16,), k, jnp.int32)
    return lax.gather(
        vec, idx[:, None],
        lax.GatherDimensionNumbers(
            offset_dims=(), collapsed_slice_dims=(0,), start_index_map=(0,)),
        slice_sizes=(1,),
        mode=lax.GatherScatterMode.PROMISE_IN_BOUNDS)


def _mm_body(x_ref, w_ref, o_ref):
    o_ref[...] = jnp.dot(x_ref[...], w_ref[...],
                         preferred_element_type=jnp.float32)


def _matmul(features, W):
    return pl.pallas_call(
        _mm_body,
        grid=(10,),
        in_specs=[
            pl.BlockSpec((N_NODES // 10, D), lambda i: (i, 0)),
            pl.BlockSpec((D, D), lambda i: (0, 0)),
        ],
        out_specs=pl.BlockSpec((N_NODES // 10, D), lambda i: (i, 0)),
        out_shape=jax.ShapeDtypeStruct((N_NODES, D), jnp.float32),
    )(features, W)


def _sc_agg_body(h_hbm, packed_hbm, ew_hbm, out_hbm,
                 acc, packed_v, ew_v, sidx, didx, rows0, rows1,
                 gsem0, gsem1, ssem0, ssem1):
    cid = lax.axis_index("c")
    sid = lax.axis_index("s")
    tid = sid * NC + cid
    rows = (rows0, rows1)
    gsem = (gsem0, gsem1)
    ssem = (ssem0, ssem1)

    def zfill(r, carry):
        for j in range(D // 16):
            rows0[r, pl.ds(j * 16, 16)] = jnp.zeros((16,), jnp.float32)
        return carry
    lax.fori_loop(0, C, zfill, 0)

    def zcopy(i, carry):
        pltpu.sync_copy(rows0, acc.at[pl.ds(sid * RPT + i * C, C)])
        return carry
    lax.fori_loop(0, RPT // C, zcopy, 0)
    pltpu.sync_copy(rows0.at[pl.ds(0, RPT % C)],
                    acc.at[pl.ds(sid * RPT + (RPT // C) * C, RPT % C)])

    @pl.when(sid == NS - 1)
    def _():
        pltpu.sync_copy(rows0.at[pl.ds(0, TAIL)],
                        acc.at[pl.ds(NS * RPT, TAIL)])

    pltpu.sync_copy(packed_hbm.at[pl.ds(tid * EPT, EPT)], packed_v)
    pltpu.sync_copy(ew_hbm.at[pl.ds(tid * EPT, EPT)], ew_v)

    def unpack(ci, b):
        for g in range(C // 16):
            p = packed_v[pl.ds(ci * C + g * 16, 16)]
            sl = pl.ds(g * 16, 16)
            sidx[b, sl] = lax.bitwise_and(p, jnp.full((16,), 0xFFFF,
                                                      jnp.int32))
            didx[b, sl] = lax.shift_right_logical(
                p, jnp.full((16,), 16, jnp.int32))

    def gissue(ci, b):
        pltpu.async_copy(h_hbm.at[sidx.at[b]], rows[b], gsem[b])

    def gwait(ci, b):
        pltpu.make_async_copy(h_hbm.at[sidx.at[b]], rows[b],
                              gsem[b]).wait()

    def sissue(ci, b):
        pltpu.async_copy(rows[b], acc.at[didx.at[b]], ssem[b], add=True)

    def swait(ci, b):
        pltpu.make_async_copy(rows[b], acc.at[didx.at[b]],
                              ssem[b]).wait()

    unpack(0, 0)
    unpack(1, 1)
    gissue(0, 0)
    gissue(1, 1)
    plsc.subcore_barrier()

    def scale(ci, b):
        def scale_group(g, carry):
            wvec = ew_v[pl.ds(ci * C + g * 16, 16)]
            for k in range(16):
                w = _vbroadcast(wvec, k)
                e = g * 16 + k
                for j in range(D // 16):
                    sl = pl.ds(j * 16, 16)
                    rows[b][e, sl] = rows[b][e, sl] * w
            return carry
        lax.fori_loop(0, C // 16, scale_group, 0)

    def process(ci, b):
        gwait(ci, b)
        scale(ci, b)
        sissue(ci, b)

    def body2(i, carry):
        ci0 = i * 2
        for b in range(2):
            ci = ci0 + b
            process(ci, b)
            swait(ci, b)

            @pl.when(ci + 2 < NCHUNK)
            def _():
                unpack(ci + 2, b)
                gissue(ci + 2, b)
        return carry
    lax.fori_loop(0, (NCHUNK - 1) // 2, body2, 0)

    process(NCHUNK - 1, 0)
    swait(NCHUNK - 1, 0)
    plsc.subcore_barrier()

    rbase = sid * RPT
    pltpu.sync_copy(acc.at[pl.ds(rbase, RPT)],
                    out_hbm.at[cid, pl.ds(rbase, RPT)])

    @pl.when(sid == NS - 1)
    def _():
        pltpu.sync_copy(acc.at[pl.ds(NS * RPT, TAIL)],
                        out_hbm.at[cid, pl.ds(NS * RPT, TAIL)])


_sc_agg = functools.partial(
    pl.kernel,
    out_type=jax.ShapeDtypeStruct((NC, N_NODES, D), jnp.float32),
    mesh=plsc.VectorSubcoreMesh(core_axis_name="c", subcore_axis_name="s"),
    scratch_types=[
        pltpu.VMEM_SHARED((N_NODES, D), jnp.float32),
        pltpu.VMEM((EPT,), jnp.int32),
        pltpu.VMEM((EPT,), jnp.float32),
        pltpu.VMEM((2, C), jnp.int32),
        pltpu.VMEM((2, C), jnp.int32),
        pltpu.VMEM((C, D), jnp.float32),
        pltpu.VMEM((C, D), jnp.float32),
        pltpu.SemaphoreType.DMA,
        pltpu.SemaphoreType.DMA,
        pltpu.SemaphoreType.DMA,
        pltpu.SemaphoreType.DMA,
    ],
)(_sc_agg_body)


def _comb_body(p0_ref, p1_ref, b_ref, o_ref):
    o_ref[...] = p0_ref[...] + p1_ref[...] + b_ref[...]


def _combine(partials, b):
    return pl.pallas_call(
        _comb_body,
        grid=(10,),
        in_specs=[
            pl.BlockSpec((N_NODES // 10, D), lambda i: (i, 0)),
            pl.BlockSpec((N_NODES // 10, D), lambda i: (i, 0)),
            pl.BlockSpec((1, D), lambda i: (0, 0)),
        ],
        out_specs=pl.BlockSpec((N_NODES // 10, D), lambda i: (i, 0)),
        out_shape=jax.ShapeDtypeStruct((N_NODES, D), jnp.float32),
    )(partials[0], partials[1], b.reshape(1, D))


@jax.jit
def kernel(features, edge_index, edge_weight, W, b):
    h = _matmul(features, W)
    packed = lax.shift_left(edge_index[1], 16) | edge_index[0]
    partials = _sc_agg(h, packed, edge_weight)
    return _combine(partials, b)


# speedup vs baseline: 11.8192x; 1.1403x over previous
import functools

import jax
import jax.numpy as jnp
from jax import lax
from jax.experimental import pallas as pl
from jax.experimental.pallas import tpu as pltpu
from jax.experimental.pallas import tpu_sc as plsc

N_NODES = 10000
N_EDGES = 320000
D = 128

NC = 2
NS = 16
NW = NC * NS

EPT = N_EDGES // NW
C = 80
NCHUNK = EPT // C
RPT = 624
TAIL = N_NODES - NS * RPT


def _vbroadcast(vec, k):
    idx = jnp.full((16,), k, jnp.int32)
    return lax.gather(
        vec, idx[:, None],
        lax.GatherDimensionNumbers(
            offset_dims=(), collapsed_slice_dims=(0,), start_index_map=(0,)),
        slice_sizes=(1,),
        mode=lax.GatherScatterMode.PROMISE_IN_BOUNDS)


def _mm_body(p0_ref, p1_ref, w_ref, b_ref, o_ref):
    o_ref[...] = jnp.dot(p0_ref[...] + p1_ref[...], w_ref[...],
                         preferred_element_type=jnp.float32) + b_ref[...]


def _combine_matmul(partials, W, b):
    return pl.pallas_call(
        _mm_body,
        grid=(10,),
        in_specs=[
            pl.BlockSpec((N_NODES // 10, D), lambda i: (i, 0)),
            pl.BlockSpec((N_NODES // 10, D), lambda i: (i, 0)),
            pl.BlockSpec((D, D), lambda i: (0, 0)),
            pl.BlockSpec((1, D), lambda i: (0, 0)),
        ],
        out_specs=pl.BlockSpec((N_NODES // 10, D), lambda i: (i, 0)),
        out_shape=jax.ShapeDtypeStruct((N_NODES, D), jnp.float32),
    )(partials[0], partials[1], W, b.reshape(1, D))


def _sc_agg_body(h_hbm, packed_hbm, ew_hbm, zeros_hbm, out_hbm,
                 acc, packed_v, sidx, didx, ew_b, rows0, rows1, rows2,
                 gsem0, gsem1, gsem2, ssem0, ssem1, ssem2,
                 esem0, esem1, esem2, zsem):
    cid = lax.axis_index("c")
    sid = lax.axis_index("s")
    tid = sid * NC + cid
    rows = (rows0, rows1, rows2)
    gsem = (gsem0, gsem1, gsem2)
    ssem = (ssem0, ssem1, ssem2)
    esem = (esem0, esem1, esem2)

    pltpu.async_copy(zeros_hbm, acc.at[pl.ds(sid * RPT, RPT)], zsem)

    @pl.when(sid == NS - 1)
    def _():
        pltpu.async_copy(zeros_hbm.at[pl.ds(0, TAIL)],
                         acc.at[pl.ds(NS * RPT, TAIL)], zsem)

    pltpu.sync_copy(packed_hbm.at[pl.ds(tid * EPT, EPT)], packed_v)

    def unpack(ci, b):
        for g in range(C // 16):
            p = packed_v[pl.ds(ci * C + g * 16, 16)]
            sl = pl.ds(g * 16, 16)
            sidx[b, sl] = lax.bitwise_and(p, jnp.full((16,), 0xFFFF,
                                                      jnp.int32))
            didx[b, sl] = lax.shift_right_logical(
                p, jnp.full((16,), 16, jnp.int32))

    def gissue(ci, b):
        pltpu.async_copy(h_hbm.at[sidx.at[b]], rows[b], gsem[b])

    def gwait(ci, b):
        pltpu.make_async_copy(h_hbm.at[sidx.at[b]], rows[b],
                              gsem[b]).wait()

    def sissue(ci, b):
        pltpu.async_copy(rows[b], acc.at[didx.at[b]], ssem[b], add=True)

    def swait(ci, b):
        pltpu.make_async_copy(rows[b], acc.at[didx.at[b]],
                              ssem[b]).wait()

    def eissue(ci, b):
        pltpu.async_copy(ew_hbm.at[pl.ds(tid * EPT + ci * C, C)],
                         ew_b.at[b], esem[b])

    def ewait(ci, b):
        pltpu.make_async_copy(ew_hbm.at[pl.ds(tid * EPT + ci * C, C)],
                              ew_b.at[b], esem[b]).wait()

    for ci in range(2):
        unpack(ci, ci)
        gissue(ci, ci)
        eissue(ci, ci)

    pltpu.make_async_copy(zeros_hbm, acc.at[pl.ds(sid * RPT, RPT)],
                          zsem).wait()

    @pl.when(sid == NS - 1)
    def _():
        pltpu.make_async_copy(zeros_hbm.at[pl.ds(0, TAIL)],
                              acc.at[pl.ds(NS * RPT, TAIL)], zsem).wait()
    plsc.subcore_barrier()

    def scale(ci, b):
        def scale_group(g, carry):
            wvec = ew_b[b, pl.ds(g * 16, 16)]
            for k in range(16):
                w = _vbroadcast(wvec, k)
                e = g * 16 + k
                for j in range(D // 16):
                    sl = pl.ds(j * 16, 16)
                    rows[b][e, sl] = rows[b][e, sl] * w
            return carry
        lax.fori_loop(0, C // 16, scale_group, 0)

    def step(ci, b, b2):
        gwait(ci, b)
        ewait(ci, b)
        scale(ci, b)

        if isinstance(ci, int):
            if ci >= 1:
                swait(ci - 1, b2)
            if ci + 2 < NCHUNK:
                unpack(ci + 2, b2)
                gissue(ci + 2, b2)
                eissue(ci + 2, b2)
        else:
            @pl.when(ci >= 1)
            def _():
                swait(ci - 1, b2)

            @pl.when(ci + 2 < NCHUNK)
            def _():
                unpack(ci + 2, b2)
                gissue(ci + 2, b2)
                eissue(ci + 2, b2)
        sissue(ci, b)

    def body3(i, carry):
        ci0 = i * 3
        for r in range(3):
            step(ci0 + r, r, (r + 2) % 3)
        return carry
    lax.fori_loop(0, NCHUNK // 3, body3, 0)

    step(NCHUNK - 2, 0, 2)
    step(NCHUNK - 1, 1, 0)
    swait(NCHUNK - 1, 1)
    plsc.subcore_barrier()

    rbase = sid * RPT
    pltpu.sync_copy(acc.at[pl.ds(rbase, RPT)],
                    out_hbm.at[cid, pl.ds(rbase, RPT)])

    @pl.when(sid == NS - 1)
    def _():
        pltpu.sync_copy(acc.at[pl.ds(NS * RPT, TAIL)],
                        out_hbm.at[cid, pl.ds(NS * RPT, TAIL)])


_sc_agg = functools.partial(
    pl.kernel,
    out_type=jax.ShapeDtypeStruct((NC, N_NODES, D), jnp.float32),
    mesh=plsc.VectorSubcoreMesh(core_axis_name="c", subcore_axis_name="s"),
    scratch_types=[
        pltpu.VMEM_SHARED((N_NODES, D), jnp.float32),
        pltpu.VMEM((EPT,), jnp.int32),
        pltpu.VMEM((3, C), jnp.int32),
        pltpu.VMEM((3, C), jnp.int32),
        pltpu.VMEM((3, C), jnp.float32),
        pltpu.VMEM((C, D), jnp.float32),
        pltpu.VMEM((C, D), jnp.float32),
        pltpu.VMEM((C, D), jnp.float32),
        pltpu.SemaphoreType.DMA,
        pltpu.SemaphoreType.DMA,
        pltpu.SemaphoreType.DMA,
        pltpu.SemaphoreType.DMA,
        pltpu.SemaphoreType.DMA,
        pltpu.SemaphoreType.DMA,
        pltpu.SemaphoreType.DMA,
        pltpu.SemaphoreType.DMA,
        pltpu.SemaphoreType.DMA,
        pltpu.SemaphoreType.DMA,
    ],
)(_sc_agg_body)


def _prep_body(ei_ref, packed_ref, zeros_ref):
    packed_ref[...] = lax.shift_left(ei_ref[1:2, :], 16) | ei_ref[0:1, :]
    zeros_ref[...] = jnp.zeros((RPT, D), jnp.float32)


def _prep(edge_index):
    return pl.pallas_call(
        _prep_body,
        grid=(4,),
        in_specs=[
            pl.BlockSpec((2, N_EDGES // 4), lambda i: (0, i)),
        ],
        out_specs=[
            pl.BlockSpec((1, N_EDGES // 4), lambda i: (0, i)),
            pl.BlockSpec((RPT, D), lambda i: (0, 0)),
        ],
        out_shape=[
            jax.ShapeDtypeStruct((1, N_EDGES), jnp.int32),
            jax.ShapeDtypeStruct((RPT, D), jnp.float32),
        ],
    )(edge_index)


@jax.jit
def kernel(features, edge_index, edge_weight, W, b):
    packed, zeros = _prep(edge_index)
    partials = _sc_agg(features, packed.reshape(N_EDGES), edge_weight, zeros)
    return _combine_matmul(partials, W, b)
